# baseline (speedup 1.0000x reference)
import jax
import jax.numpy as jnp
from jax.experimental import pallas as pl

B = 16384
FD = 64
CTX = 128
RD = 256
BB = 4096

_DN = (((0,), (0,)), ((), ()))


def _mlp_body(idx_ref, mol_ref, tt_ref, pt_ref, gt_ref, rt_ref,
              w1_ref, b1_ref, w2_ref, b2_ref, out_ref):
    f32 = jnp.float32
    iota16 = jax.lax.broadcasted_iota(jnp.int32, (16, BB), 0)
    iota8 = jax.lax.broadcasted_iota(jnp.int32, (8, BB), 0)
    ohT_t = (idx_ref[0:1, :] == iota16).astype(f32)
    ohT_p = (idx_ref[1:2, :] == iota8).astype(f32)
    ohT_g = (idx_ref[2:3, :] == iota8).astype(f32)
    ohT_r = (idx_ref[3:4, :] == iota8).astype(f32)

    p_t = jnp.dot(tt_ref[...], w1_ref[0:64, :], preferred_element_type=f32)
    p_p = jnp.dot(pt_ref[...], w1_ref[64:128, :], preferred_element_type=f32)
    p_g = jnp.dot(gt_ref[...], w1_ref[128:192, :], preferred_element_type=f32)
    p_r = jnp.dot(rt_ref[...], w1_ref[192:256, :], preferred_element_type=f32)

    dg = lambda a, b: jax.lax.dot_general(a, b, _DN, preferred_element_type=f32)
    acc = jnp.dot(mol_ref[...], w1_ref[257:513, :], preferred_element_type=f32)
    acc = acc + dg(ohT_t, p_t)
    acc = acc + dg(ohT_p, p_p)
    acc = acc + dg(ohT_g, p_g)
    acc = acc + dg(ohT_r, p_r)
    logit_row = jax.lax.bitcast_convert_type(idx_ref[4:5, :], jnp.float32)
    acc = acc + dg(logit_row, w1_ref[256:257, :])
    acc = acc + b1_ref[...]
    h = 0.5 * acc * (1.0 + jax.lax.erf(acc * 0.7071067811865476))
    out_ref[...] = jnp.dot(h, w2_ref[...], preferred_element_type=f32) + b2_ref[...]


@jax.jit
def _run(idx5, mol_repr, type_table, prep_table, geom_table,
         read_table, w1_full, b1_2d, w2, b2_2d):
    nb = B // BB
    full = lambda shape: pl.BlockSpec(shape, lambda i: (0, 0))
    return pl.pallas_call(
        _mlp_body,
        grid=(nb,),
        in_specs=[
            pl.BlockSpec((5, BB), lambda i: (0, i)),
            pl.BlockSpec((BB, RD), lambda i: (i, 0)),
            full((16, FD)), full((8, FD)), full((8, FD)), full((8, FD)),
            full((4 * FD + 1 + RD, CTX)),
            full((1, CTX)), full((CTX, CTX)), full((1, CTX)),
        ],
        out_specs=pl.BlockSpec((BB, CTX), lambda i: (i, 0)),
        out_shape=jax.ShapeDtypeStruct((B, CTX), jnp.float32),
    )(idx5, mol_repr, type_table, prep_table, geom_table,
      read_table, w1_full, b1_2d, w2, b2_2d)


def kernel(assay_type_idx, assay_prep_idx, assay_geometry_idx, assay_readout_idx,
           binding_logit, mol_repr, type_table, prep_table, geom_table, read_table,
           W1, b1, W2, b2):
    i32 = jnp.int32
    idx5 = jnp.stack(
        [assay_type_idx.astype(i32), assay_prep_idx.astype(i32),
         assay_geometry_idx.astype(i32), assay_readout_idx.astype(i32),
         jax.lax.bitcast_convert_type(binding_logit, i32)], axis=0)
    return _run(idx5, mol_repr, type_table, prep_table, geom_table,
                read_table, W1, b1.reshape(1, CTX), W2, b2.reshape(1, CTX))


# speedup vs baseline: 1.0677x; 1.0677x over previous
import jax
import jax.numpy as jnp
from jax.experimental import pallas as pl

B = 16384
FD = 64
CTX = 128
RD = 256
BB = 4096

_DN = (((0,), (0,)), ((), ()))


def _mlp_body(idx_ref, logit_ref, mol_ref, tt_ref, pt_ref, gt_ref, rt_ref,
              w1_ref, b1_ref, w2_ref, b2_ref, out_ref):
    f32 = jnp.float32
    iota16 = jax.lax.broadcasted_iota(jnp.int32, (16, BB), 0)
    iota8 = jax.lax.broadcasted_iota(jnp.int32, (8, BB), 0)
    ohT_t = (idx_ref[0:1, :] == iota16).astype(f32)
    ohT_p = (idx_ref[1:2, :] == iota8).astype(f32)
    ohT_g = (idx_ref[2:3, :] == iota8).astype(f32)
    ohT_r = (idx_ref[3:4, :] == iota8).astype(f32)

    p_t = jnp.dot(tt_ref[...], w1_ref[0:64, :], preferred_element_type=f32)
    p_p = jnp.dot(pt_ref[...], w1_ref[64:128, :], preferred_element_type=f32)
    p_g = jnp.dot(gt_ref[...], w1_ref[128:192, :], preferred_element_type=f32)
    p_r = jnp.dot(rt_ref[...], w1_ref[192:256, :], preferred_element_type=f32)

    dg = lambda a, b: jax.lax.dot_general(a, b, _DN, preferred_element_type=f32)
    acc = jnp.dot(mol_ref[...], w1_ref[257:513, :], preferred_element_type=f32)
    acc = acc + dg(ohT_t, p_t)
    acc = acc + dg(ohT_p, p_p)
    acc = acc + dg(ohT_g, p_g)
    acc = acc + dg(ohT_r, p_r)
    acc = acc + dg(logit_ref[...], w1_ref[256:257, :])
    acc = acc + b1_ref[...]
    h = 0.5 * acc * (1.0 + jax.lax.erf(acc * 0.7071067811865476))
    out_ref[...] = jnp.dot(h, w2_ref[...], preferred_element_type=f32) + b2_ref[...]


@jax.jit
def _run(idx4, logit_row, mol_repr, type_table, prep_table, geom_table,
         read_table, w1_full, b1_2d, w2, b2_2d):
    nb = B // BB
    full = lambda shape: pl.BlockSpec(shape, lambda i: (0, 0))
    return pl.pallas_call(
        _mlp_body,
        grid=(nb,),
        in_specs=[
            pl.BlockSpec((4, BB), lambda i: (0, i)),
            pl.BlockSpec((1, BB), lambda i: (0, i)),
            pl.BlockSpec((BB, RD), lambda i: (i, 0)),
            full((16, FD)), full((8, FD)), full((8, FD)), full((8, FD)),
            full((4 * FD + 1 + RD, CTX)),
            full((1, CTX)), full((CTX, CTX)), full((1, CTX)),
        ],
        out_specs=pl.BlockSpec((BB, CTX), lambda i: (i, 0)),
        out_shape=jax.ShapeDtypeStruct((B, CTX), jnp.float32),
    )(idx4, logit_row, mol_repr, type_table, prep_table, geom_table,
      read_table, w1_full, b1_2d, w2, b2_2d)


def kernel(assay_type_idx, assay_prep_idx, assay_geometry_idx, assay_readout_idx,
           binding_logit, mol_repr, type_table, prep_table, geom_table, read_table,
           W1, b1, W2, b2):
    i32 = jnp.int32
    idx4 = jnp.stack(
        [assay_type_idx.astype(i32), assay_prep_idx.astype(i32),
         assay_geometry_idx.astype(i32), assay_readout_idx.astype(i32)], axis=0)
    logit_row = binding_logit.reshape(1, B)
    return _run(idx4, logit_row, mol_repr, type_table, prep_table, geom_table,
                read_table, W1, b1.reshape(1, CTX), W2, b2.reshape(1, CTX))
